# Initial kernel scaffold; baseline (speedup 1.0000x reference)
import jax
import jax.numpy as jnp
from jax.experimental import pallas as pl


def kernel(words, table):
    raise NotImplementedError("write your pallas kernel here")


# speedup vs baseline: 1.6378x; 1.6378x over previous
import functools

import jax
import jax.numpy as jnp
import numpy as np
from jax import lax
from jax.experimental import pallas as pl
from jax.experimental.pallas import tpu as pltpu
from jax.experimental.pallas import tpu_sc as plsc

NUM_EMB = 1000000
D = 64
P_DROP = 0.1
B = 16384 * 50

NC = 2
NS = 16
L = 16
NW = NC * NS
B_PER_W = B // NW
C = 512
N_CHUNKS = B_PER_W // C

BITS_WORDS = 32768
INV_KEEP = float(np.float32(1.0) / np.float32(1.0 - P_DROP))


@functools.partial(
    pl.kernel,
    mesh=plsc.VectorSubcoreMesh(core_axis_name="c", subcore_axis_name="s"),
    out_type=jax.ShapeDtypeStruct((B, D), jnp.float32),
    compiler_params=pltpu.CompilerParams(
        needs_layout_passes=False, use_tc_tiling_on_sc=False),
    scratch_types=[
        pltpu.VMEM((BITS_WORDS,), jnp.int32),
        pltpu.VMEM((C,), jnp.int32),
        pltpu.VMEM((C,), jnp.float32),
        pltpu.VMEM((C, D), jnp.float32),
        pltpu.SemaphoreType.DMA,
    ],
)
def _emb_dropout_gather(table_hbm, idx_hbm, bits_hbm, out_hbm,
                        bits_v, idx_v, scale_v, rows_v, sem):
    wid = lax.axis_index("s") * NC + lax.axis_index("c")
    base = wid * B_PER_W
    pltpu.sync_copy(bits_hbm, bits_v)

    def chunk_body(c, carry):
        off = base + c * C
        pltpu.sync_copy(idx_hbm.at[pl.ds(off, C)], idx_v)
        pltpu.async_copy(table_hbm.at[idx_v], rows_v, sem).wait()

        def scale_body(j, carry2):
            idx16 = idx_v[pl.ds(j * L, L)]
            w = lax.shift_right_logical(idx16, 5)
            bpos = lax.bitwise_and(idx16, 31)
            word = plsc.load_gather(bits_v, [w])
            bit = lax.bitwise_and(lax.shift_right_logical(word, bpos), 1)
            scale_v[pl.ds(j * L, L)] = bit.astype(jnp.float32) * INV_KEEP
            return carry2

        lax.fori_loop(0, C // L, scale_body, 0)

        def mul_body(b, carry2):
            bvec = jnp.zeros((L,), jnp.int32) + b
            s = plsc.load_gather(scale_v, [bvec])
            for k in range(D // L):
                rows_v[b, pl.ds(k * L, L)] = rows_v[b, pl.ds(k * L, L)] * s
            return carry2

        lax.fori_loop(0, C, mul_body, 0)

        pltpu.sync_copy(rows_v, out_hbm.at[pl.ds(off, C)])
        return carry

    lax.fori_loop(0, N_CHUNKS, chunk_body, 0)


def _pack_keep_bits():
    keep = jax.random.bernoulli(
        jax.random.key(42), 1.0 - P_DROP, (NUM_EMB, 1))
    kb = keep[:, 0]
    kb = jnp.pad(kb, (0, BITS_WORDS * 32 - NUM_EMB))
    kw = kb.reshape(BITS_WORDS, 32).astype(jnp.uint32)
    shifts = jnp.arange(32, dtype=jnp.uint32)[None, :]
    words_u = jnp.sum(kw << shifts, axis=1, dtype=jnp.uint32)
    return lax.bitcast_convert_type(words_u, jnp.int32)


def kernel(words, table):
    bits = _pack_keep_bits()
    idx = words.reshape(B)
    out = _emb_dropout_gather(table, idx, bits)
    return out.reshape(words.shape[0], words.shape[1], D)
